# Initial kernel scaffold; baseline (speedup 1.0000x reference)
import functools

import jax
import jax.numpy as jnp
from jax import lax
from jax.experimental import pallas as pl
from jax.experimental.pallas import tpu as pltpu
from jax.experimental.pallas import tpu_sc as plsc

N = 10000
E = 320000
NC = 2
NS = 16
LANE = 16
CHUNK = 128
NCHUNKS = E // CHUNK
ROWS_PER_TILE = N // NS
ZROWS = 125
RSPAD = 10240
RS_PER_TILE = RSPAD // NS

_SC_MESH = plsc.VectorSubcoreMesh(core_axis_name="c", subcore_axis_name="s")
_SC_PARAMS = pltpu.CompilerParams(needs_layout_passes=False)


def _make_edge_kernel(D):
    @functools.partial(
        pl.kernel,
        out_type=[
            jax.ShapeDtypeStruct((NC, N, D), jnp.float32),
            jax.ShapeDtypeStruct((NC, RSPAD), jnp.float32),
        ],
        mesh=_SC_MESH,
        compiler_params=_SC_PARAMS,
        scratch_types=[
            pltpu.VMEM((N,), jnp.float32),
            pltpu.VMEM((N,), jnp.float32),
            pltpu.VMEM((1, CHUNK), jnp.int32),
            pltpu.VMEM((1, CHUNK), jnp.int32),
            pltpu.VMEM((CHUNK,), jnp.float32),
            pltpu.VMEM((CHUNK,), jnp.float32),
            pltpu.VMEM((CHUNK, D), jnp.float32),
            pltpu.VMEM((ZROWS, D), jnp.float32),
            pltpu.VMEM((RS_PER_TILE,), jnp.float32),
            pltpu.VMEM_SHARED((N, D), jnp.float32),
            pltpu.VMEM_SHARED((RSPAD,), jnp.float32),
            pltpu.SemaphoreType.DMA,
        ],
    )
    def edge_kernel(src_hbm, dst_hbm, a_hbm, h_hbm, f1_hbm, f2_hbm,
                    u_out, rs_out,
                    f1_v, f2_v, src_v, dst_v, a_v, w_v, rows_v,
                    zrow_v, zrs_v, u_sh, rs_sh, sem):
        c = lax.axis_index("c")
        s = lax.axis_index("s")
        wid = c * NS + s

        zero16 = jnp.zeros((LANE,), jnp.float32)

        @pl.loop(0, ZROWS)
        def _(i):
            @pl.loop(0, D, step=LANE)
            def _(q):
                zrow_v[i, pl.ds(q, LANE)] = zero16

        @pl.loop(0, RS_PER_TILE, step=LANE)
        def _(q):
            zrs_v[pl.ds(q, LANE)] = zero16

        @pl.loop(0, ROWS_PER_TILE // ZROWS)
        def _(k):
            pltpu.sync_copy(zrow_v, u_sh.at[pl.ds(s * ROWS_PER_TILE + k * ZROWS, ZROWS)])

        pltpu.sync_copy(zrs_v, rs_sh.at[pl.ds(s * RS_PER_TILE, RS_PER_TILE)])

        pltpu.sync_copy(f1_hbm, f1_v)
        pltpu.sync_copy(f2_hbm, f2_v)

        plsc.subcore_barrier()

        @pl.loop(wid, NCHUNKS, step=NC * NS)
        def _(t):
            e0 = t * CHUNK
            pltpu.sync_copy(src_hbm.at[pl.ds(e0, CHUNK)], src_v.at[0])
            pltpu.sync_copy(dst_hbm.at[pl.ds(e0, CHUNK)], dst_v.at[0])
            pltpu.sync_copy(a_hbm.at[pl.ds(e0, CHUNK)], a_v)
            pltpu.async_copy(h_hbm.at[dst_v.at[0]], rows_v, sem).wait()

            for g in range(CHUNK // LANE):
                s16 = src_v[0, pl.ds(g * LANE, LANE)]
                d16 = dst_v[0, pl.ds(g * LANE, LANE)]
                a16 = a_v[pl.ds(g * LANE, LANE)]
                f1g = plsc.load_gather(f1_v, [s16])
                f2g = plsc.load_gather(f2_v, [d16])
                x = a16 * (f1g + f2g)
                att = 1.0 / (1.0 + jnp.exp(-x))
                w16 = jnp.exp(att)
                w_v[pl.ds(g * LANE, LANE)] = w16
                for j in range(LANE):
                    wj = jnp.take(w16, jnp.full((LANE,), j, jnp.int32))
                    r = g * LANE + j
                    for q in range(D // LANE):
                        rows_v[r, pl.ds(q * LANE, LANE)] = (
                            rows_v[r, pl.ds(q * LANE, LANE)] * wj)

            pltpu.sync_copy(w_v, rs_sh.at[src_v.at[0]], add=True)
            pltpu.sync_copy(rows_v, u_sh.at[src_v.at[0]], add=True)

        plsc.subcore_barrier()

        @pl.loop(0, ROWS_PER_TILE // ZROWS)
        def _(k):
            r0 = s * ROWS_PER_TILE + k * ZROWS
            pltpu.sync_copy(u_sh.at[pl.ds(r0, ZROWS)], u_out.at[c, pl.ds(r0, ZROWS)])

        pltpu.sync_copy(rs_sh.at[pl.ds(s * RS_PER_TILE, RS_PER_TILE)],
                        rs_out.at[c, pl.ds(s * RS_PER_TILE, RS_PER_TILE)])

    return edge_kernel


_edge_kernel_64 = _make_edge_kernel(64)
_edge_kernel_32 = _make_edge_kernel(32)


def _dot(a, b):
    return lax.dot_general(a, b, (((1,), (0,)), ((), ())),
                           preferred_element_type=jnp.float32,
                           precision=lax.Precision.HIGHEST)


def _encode1_body(x_ref, w_ref, v0_ref, v1_ref, h_ref, f1_ref, f2_ref):
    h = _dot(x_ref[...], w_ref[...])
    h_ref[...] = h
    f1_ref[...] = _dot(h, v0_ref[...])
    f2_ref[...] = _dot(h, v1_ref[...])


def _encode1(X, W0, v0, v1):
    bm = 1000
    return pl.pallas_call(
        _encode1_body,
        grid=(N // bm,),
        in_specs=[
            pl.BlockSpec((bm, 128), lambda i: (i, 0)),
            pl.BlockSpec((128, 64), lambda i: (0, 0)),
            pl.BlockSpec((64, 1), lambda i: (0, 0)),
            pl.BlockSpec((64, 1), lambda i: (0, 0)),
        ],
        out_specs=[
            pl.BlockSpec((bm, 64), lambda i: (i, 0)),
            pl.BlockSpec((bm, 1), lambda i: (i, 0)),
            pl.BlockSpec((bm, 1), lambda i: (i, 0)),
        ],
        out_shape=[
            jax.ShapeDtypeStruct((N, 64), jnp.float32),
            jax.ShapeDtypeStruct((N, 1), jnp.float32),
            jax.ShapeDtypeStruct((N, 1), jnp.float32),
        ],
    )(X, W0, v0, v1)


def _combine_encode2_body(u0_ref, u1_ref, r0_ref, r1_ref, w_ref, v0_ref,
                          v1_ref, h_ref, f1_ref, f2_ref):
    rs = r0_ref[...] + r1_ref[...]
    rs = jnp.where(rs == 0.0, 1.0, rs)
    hin = (u0_ref[...] + u1_ref[...]) / rs
    h = _dot(hin, w_ref[...])
    h_ref[...] = h
    f1_ref[...] = _dot(h, v0_ref[...])
    f2_ref[...] = _dot(h, v1_ref[...])


def _combine_encode2(U0, U1, r0, r1, W1, v0, v1):
    bm = 1000
    return pl.pallas_call(
        _combine_encode2_body,
        grid=(N // bm,),
        in_specs=[
            pl.BlockSpec((bm, 64), lambda i: (i, 0)),
            pl.BlockSpec((bm, 64), lambda i: (i, 0)),
            pl.BlockSpec((bm, 1), lambda i: (i, 0)),
            pl.BlockSpec((bm, 1), lambda i: (i, 0)),
            pl.BlockSpec((64, 32), lambda i: (0, 0)),
            pl.BlockSpec((32, 1), lambda i: (0, 0)),
            pl.BlockSpec((32, 1), lambda i: (0, 0)),
        ],
        out_specs=[
            pl.BlockSpec((bm, 32), lambda i: (i, 0)),
            pl.BlockSpec((bm, 1), lambda i: (i, 0)),
            pl.BlockSpec((bm, 1), lambda i: (i, 0)),
        ],
        out_shape=[
            jax.ShapeDtypeStruct((N, 32), jnp.float32),
            jax.ShapeDtypeStruct((N, 1), jnp.float32),
            jax.ShapeDtypeStruct((N, 1), jnp.float32),
        ],
    )(U0, U1, r0, r1, W1, v0, v1)


def _combine_body(u0_ref, u1_ref, r0_ref, r1_ref, h_ref):
    rs = r0_ref[...] + r1_ref[...]
    rs = jnp.where(rs == 0.0, 1.0, rs)
    h_ref[...] = (u0_ref[...] + u1_ref[...]) / rs


def _combine(U0, U1, r0, r1):
    bm = 1000
    return pl.pallas_call(
        _combine_body,
        grid=(N // bm,),
        in_specs=[
            pl.BlockSpec((bm, 32), lambda i: (i, 0)),
            pl.BlockSpec((bm, 32), lambda i: (i, 0)),
            pl.BlockSpec((bm, 1), lambda i: (i, 0)),
            pl.BlockSpec((bm, 1), lambda i: (i, 0)),
        ],
        out_specs=pl.BlockSpec((bm, 32), lambda i: (i, 0)),
        out_shape=jax.ShapeDtypeStruct((N, 32), jnp.float32),
    )(U0, U1, r0, r1)


def _decode_body(a_ref, b_ref, o_ref):
    z = lax.dot_general(a_ref[...], b_ref[...], (((1,), (1,)), ((), ())),
                        preferred_element_type=jnp.float32,
                        precision=lax.Precision.HIGHEST)
    o_ref[...] = jax.nn.sigmoid(z)


def _decode(Hf):
    bm = 512
    g = pl.cdiv(N, bm)
    return pl.pallas_call(
        _decode_body,
        grid=(g, g),
        in_specs=[
            pl.BlockSpec((bm, 32), lambda i, j: (i, 0)),
            pl.BlockSpec((bm, 32), lambda i, j: (j, 0)),
        ],
        out_specs=pl.BlockSpec((bm, bm), lambda i, j: (i, j)),
        out_shape=jax.ShapeDtypeStruct((N, N), jnp.float32),
        compiler_params=pltpu.CompilerParams(
            dimension_semantics=("parallel", "parallel")),
    )(Hf)


def kernel(X, edge_index, A_values, W0, W1, v0_0, v1_0, v0_1, v1_1):
    src = edge_index[0]
    dst = edge_index[1]

    H1, f1a, f2a = _encode1(X, W0, v0_0, v1_0)
    U1, RS1 = _edge_kernel_64(src, dst, A_values, H1,
                              f1a.reshape(N), f2a.reshape(N))
    H2, f1b, f2b = _combine_encode2(
        U1[0], U1[1],
        RS1[0, :N].reshape(N, 1), RS1[1, :N].reshape(N, 1),
        W1, v0_1, v1_1)
    U2, RS2 = _edge_kernel_32(src, dst, A_values, H2,
                              f1b.reshape(N), f2b.reshape(N))
    Hf = _combine(U2[0], U2[1],
                  RS2[0, :N].reshape(N, 1), RS2[1, :N].reshape(N, 1))
    return _decode(Hf)


# speedup vs baseline: 13.8088x; 13.8088x over previous
import functools

import jax
import jax.numpy as jnp
from jax import lax
from jax.experimental import pallas as pl
from jax.experimental.pallas import tpu as pltpu
from jax.experimental.pallas import tpu_sc as plsc

N = 10000
E = 320000
NC = 2
NS = 16
LANE = 16
CHUNK = 128
NCHUNKS = E // CHUNK
NPAD = 10240
ROWS_PER_TILE = NPAD // NS
ZROWS = 128

_SC_MESH = plsc.VectorSubcoreMesh(core_axis_name="c", subcore_axis_name="s")
_SC_PARAMS = pltpu.CompilerParams(needs_layout_passes=False,
                                  use_tc_tiling_on_sc=False)


def _make_edge_kernel(D):
    @functools.partial(
        pl.kernel,
        out_type=[
            jax.ShapeDtypeStruct((NC, NPAD, D), jnp.float32),
            jax.ShapeDtypeStruct((NC, 1, NPAD), jnp.float32),
        ],
        mesh=_SC_MESH,
        compiler_params=_SC_PARAMS,
        scratch_types=[
            pltpu.VMEM((N,), jnp.float32),
            pltpu.VMEM((N,), jnp.float32),
            pltpu.VMEM((1, CHUNK), jnp.int32),
            pltpu.VMEM((1, CHUNK), jnp.int32),
            pltpu.VMEM((CHUNK,), jnp.float32),
            pltpu.VMEM((CHUNK,), jnp.float32),
            pltpu.VMEM((CHUNK, D), jnp.float32),
            pltpu.VMEM((ZROWS, D), jnp.float32),
            pltpu.VMEM((ROWS_PER_TILE,), jnp.float32),
            pltpu.VMEM_SHARED((NPAD, D), jnp.float32),
            pltpu.VMEM_SHARED((NPAD,), jnp.float32),
            pltpu.SemaphoreType.DMA,
        ],
    )
    def edge_kernel(src_hbm, dst_hbm, a_hbm, h_hbm, f1_hbm, f2_hbm,
                    u_out, rs_out,
                    f1_v, f2_v, src_v, dst_v, a_v, w_v, rows_v,
                    zrow_v, zrs_v, u_sh, rs_sh, sem):
        c = lax.axis_index("c")
        s = lax.axis_index("s")
        wid = c * NS + s

        zero16 = jnp.zeros((LANE,), jnp.float32)

        @pl.loop(0, ZROWS)
        def _(i):
            @pl.loop(0, D, step=LANE)
            def _(q):
                zrow_v[i, pl.ds(q, LANE)] = zero16

        @pl.loop(0, ROWS_PER_TILE, step=LANE)
        def _(q):
            zrs_v[pl.ds(q, LANE)] = zero16

        @pl.loop(0, ROWS_PER_TILE // ZROWS)
        def _(k):
            pltpu.sync_copy(zrow_v, u_sh.at[pl.ds(s * ROWS_PER_TILE + k * ZROWS, ZROWS)])

        pltpu.sync_copy(zrs_v, rs_sh.at[pl.ds(s * ROWS_PER_TILE, ROWS_PER_TILE)])

        pltpu.sync_copy(f1_hbm, f1_v)
        pltpu.sync_copy(f2_hbm, f2_v)

        plsc.subcore_barrier()

        @pl.loop(wid, NCHUNKS, step=NC * NS)
        def _(t):
            e0 = t * CHUNK
            pltpu.sync_copy(src_hbm.at[pl.ds(e0, CHUNK)], src_v.at[0])
            pltpu.sync_copy(dst_hbm.at[pl.ds(e0, CHUNK)], dst_v.at[0])
            pltpu.sync_copy(a_hbm.at[pl.ds(e0, CHUNK)], a_v)
            pltpu.async_copy(h_hbm.at[dst_v.at[0]], rows_v, sem).wait()

            for g in range(CHUNK // LANE):
                s16 = src_v[0, pl.ds(g * LANE, LANE)]
                d16 = dst_v[0, pl.ds(g * LANE, LANE)]
                a16 = a_v[pl.ds(g * LANE, LANE)]
                f1g = plsc.load_gather(f1_v, [s16])
                f2g = plsc.load_gather(f2_v, [d16])
                x = a16 * (f1g + f2g)
                att = 1.0 / (1.0 + jnp.exp(-x))
                w16 = jnp.exp(att)
                w_v[pl.ds(g * LANE, LANE)] = w16
                for j in range(LANE):
                    wj = jnp.take(w16, jnp.full((LANE,), j, jnp.int32))
                    r = g * LANE + j
                    for q in range(D // LANE):
                        rows_v[r, pl.ds(q * LANE, LANE)] = (
                            rows_v[r, pl.ds(q * LANE, LANE)] * wj)

            pltpu.sync_copy(w_v, rs_sh.at[src_v.at[0]], add=True)
            pltpu.sync_copy(rows_v, u_sh.at[src_v.at[0]], add=True)

        plsc.subcore_barrier()

        @pl.loop(0, ROWS_PER_TILE // ZROWS)
        def _(k):
            r0 = s * ROWS_PER_TILE + k * ZROWS
            pltpu.sync_copy(u_sh.at[pl.ds(r0, ZROWS)], u_out.at[c, pl.ds(r0, ZROWS)])

        pltpu.sync_copy(rs_sh.at[pl.ds(s * ROWS_PER_TILE, ROWS_PER_TILE)],
                        rs_out.at[c, 0, pl.ds(s * ROWS_PER_TILE, ROWS_PER_TILE)])

    return edge_kernel


_edge_kernel_64 = _make_edge_kernel(64)
_edge_kernel_32 = _make_edge_kernel(32)


def _dot(a, b):
    return lax.dot_general(a, b, (((1,), (0,)), ((), ())),
                           preferred_element_type=jnp.float32,
                           precision=lax.Precision.HIGHEST)


def _encode1_body(x_ref, w_ref, v0_ref, v1_ref, h_ref, f1_ref, f2_ref):
    h = _dot(x_ref[...], w_ref[...])
    h_ref[...] = h
    f1_ref[...] = _dot(h, v0_ref[...])
    f2_ref[...] = _dot(h, v1_ref[...])


def _encode1(X, W0, v0, v1):
    bm = 1000
    return pl.pallas_call(
        _encode1_body,
        grid=(N // bm,),
        in_specs=[
            pl.BlockSpec((bm, 128), lambda i: (i, 0)),
            pl.BlockSpec((128, 64), lambda i: (0, 0)),
            pl.BlockSpec((64, 1), lambda i: (0, 0)),
            pl.BlockSpec((64, 1), lambda i: (0, 0)),
        ],
        out_specs=[
            pl.BlockSpec((bm, 64), lambda i: (i, 0)),
            pl.BlockSpec((bm, 1), lambda i: (i, 0)),
            pl.BlockSpec((bm, 1), lambda i: (i, 0)),
        ],
        out_shape=[
            jax.ShapeDtypeStruct((N, 64), jnp.float32),
            jax.ShapeDtypeStruct((N, 1), jnp.float32),
            jax.ShapeDtypeStruct((N, 1), jnp.float32),
        ],
    )(X, W0, v0, v1)


def _combine_encode2_body(u0_ref, u1_ref, r0_ref, r1_ref, w_ref, v0_ref,
                          v1_ref, h_ref, f1_ref, f2_ref):
    rs = r0_ref[...] + r1_ref[...]
    rs = jnp.where(rs == 0.0, 1.0, rs)
    hin = (u0_ref[...] + u1_ref[...]) / rs
    h = _dot(hin, w_ref[...])
    h_ref[...] = h
    f1_ref[...] = _dot(h, v0_ref[...])
    f2_ref[...] = _dot(h, v1_ref[...])


def _combine_encode2(U0, U1, r0, r1, W1, v0, v1):
    bm = 1000
    return pl.pallas_call(
        _combine_encode2_body,
        grid=(N // bm,),
        in_specs=[
            pl.BlockSpec((bm, 64), lambda i: (i, 0)),
            pl.BlockSpec((bm, 64), lambda i: (i, 0)),
            pl.BlockSpec((bm, 1), lambda i: (i, 0)),
            pl.BlockSpec((bm, 1), lambda i: (i, 0)),
            pl.BlockSpec((64, 32), lambda i: (0, 0)),
            pl.BlockSpec((32, 1), lambda i: (0, 0)),
            pl.BlockSpec((32, 1), lambda i: (0, 0)),
        ],
        out_specs=[
            pl.BlockSpec((bm, 32), lambda i: (i, 0)),
            pl.BlockSpec((bm, 1), lambda i: (i, 0)),
            pl.BlockSpec((bm, 1), lambda i: (i, 0)),
        ],
        out_shape=[
            jax.ShapeDtypeStruct((N, 32), jnp.float32),
            jax.ShapeDtypeStruct((N, 1), jnp.float32),
            jax.ShapeDtypeStruct((N, 1), jnp.float32),
        ],
    )(U0, U1, r0, r1, W1, v0, v1)


def _combine_body(u0_ref, u1_ref, r0_ref, r1_ref, h_ref):
    rs = r0_ref[...] + r1_ref[...]
    rs = jnp.where(rs == 0.0, 1.0, rs)
    h_ref[...] = (u0_ref[...] + u1_ref[...]) / rs


def _combine(U0, U1, r0, r1):
    bm = 1000
    return pl.pallas_call(
        _combine_body,
        grid=(N // bm,),
        in_specs=[
            pl.BlockSpec((bm, 32), lambda i: (i, 0)),
            pl.BlockSpec((bm, 32), lambda i: (i, 0)),
            pl.BlockSpec((bm, 1), lambda i: (i, 0)),
            pl.BlockSpec((bm, 1), lambda i: (i, 0)),
        ],
        out_specs=pl.BlockSpec((bm, 32), lambda i: (i, 0)),
        out_shape=jax.ShapeDtypeStruct((N, 32), jnp.float32),
    )(U0, U1, r0, r1)


def _decode_body(a_ref, b_ref, o_ref):
    z = lax.dot_general(a_ref[...], b_ref[...], (((1,), (1,)), ((), ())),
                        preferred_element_type=jnp.float32,
                        precision=lax.Precision.HIGHEST)
    o_ref[...] = jax.nn.sigmoid(z)


def _decode(Hf):
    bm = 512
    g = pl.cdiv(N, bm)
    return pl.pallas_call(
        _decode_body,
        grid=(g, g),
        in_specs=[
            pl.BlockSpec((bm, 32), lambda i, j: (i, 0)),
            pl.BlockSpec((bm, 32), lambda i, j: (j, 0)),
        ],
        out_specs=pl.BlockSpec((bm, bm), lambda i, j: (i, j)),
        out_shape=jax.ShapeDtypeStruct((N, N), jnp.float32),
        compiler_params=pltpu.CompilerParams(
            dimension_semantics=("parallel", "parallel")),
    )(Hf, Hf)


def kernel(X, edge_index, A_values, W0, W1, v0_0, v1_0, v0_1, v1_1):
    src = edge_index[0]
    dst = edge_index[1]

    H1, f1a, f2a = _encode1(X, W0, v0_0, v1_0)
    U1, RS1 = _edge_kernel_64(src, dst, A_values, H1,
                              f1a.reshape(N), f2a.reshape(N))
    H2, f1b, f2b = _combine_encode2(
        U1[0, :N], U1[1, :N],
        RS1[0, 0, :N].reshape(N, 1), RS1[1, 0, :N].reshape(N, 1),
        W1, v0_1, v1_1)
    U2, RS2 = _edge_kernel_32(src, dst, A_values, H2,
                              f1b.reshape(N), f2b.reshape(N))
    Hf = _combine(U2[0, :N], U2[1, :N],
                  RS2[0, 0, :N].reshape(N, 1), RS2[1, 0, :N].reshape(N, 1))
    return _decode(Hf)


# speedup vs baseline: 15.9049x; 1.1518x over previous
import functools

import jax
import jax.numpy as jnp
from jax import lax
from jax.experimental import pallas as pl
from jax.experimental.pallas import tpu as pltpu
from jax.experimental.pallas import tpu_sc as plsc

N = 10000
E = 320000
NC = 2
NS = 16
LANE = 16
CHUNK = 128
NCHUNKS = E // CHUNK
NPAD = 10240
ROWS_PER_TILE = NPAD // NS
ZROWS = 128

_SC_MESH = plsc.VectorSubcoreMesh(core_axis_name="c", subcore_axis_name="s")
_SC_PARAMS = pltpu.CompilerParams(needs_layout_passes=False,
                                  use_tc_tiling_on_sc=False)


def _make_edge_kernel(D):
    @functools.partial(
        pl.kernel,
        out_type=[
            jax.ShapeDtypeStruct((NC, NPAD, D), jnp.float32),
            jax.ShapeDtypeStruct((NC, 1, NPAD), jnp.float32),
        ],
        mesh=_SC_MESH,
        compiler_params=_SC_PARAMS,
        scratch_types=[
            pltpu.VMEM((N,), jnp.float32),
            pltpu.VMEM((N,), jnp.float32),
            pltpu.VMEM((1, CHUNK), jnp.int32),
            pltpu.VMEM((1, CHUNK), jnp.int32),
            pltpu.VMEM((CHUNK,), jnp.float32),
            pltpu.VMEM((CHUNK,), jnp.float32),
            pltpu.VMEM((CHUNK, D), jnp.float32),
            pltpu.VMEM((ZROWS, D), jnp.float32),
            pltpu.VMEM((ROWS_PER_TILE,), jnp.float32),
            pltpu.VMEM_SHARED((NPAD, D), jnp.float32),
            pltpu.VMEM_SHARED((NPAD,), jnp.float32),
            pltpu.SemaphoreType.DMA,
        ],
    )
    def edge_kernel(src_hbm, dst_hbm, a_hbm, h_hbm, f1_hbm, f2_hbm,
                    u_out, rs_out,
                    f1_v, f2_v, src_v, dst_v, a_v, w_v, rows_v,
                    zrow_v, zrs_v, u_sh, rs_sh, sem):
        c = lax.axis_index("c")
        s = lax.axis_index("s")
        wid = c * NS + s

        zero16 = jnp.zeros((LANE,), jnp.float32)

        @pl.loop(0, ZROWS)
        def _(i):
            @pl.loop(0, D, step=LANE)
            def _(q):
                zrow_v[i, pl.ds(q, LANE)] = zero16

        @pl.loop(0, ROWS_PER_TILE, step=LANE)
        def _(q):
            zrs_v[pl.ds(q, LANE)] = zero16

        @pl.loop(0, ROWS_PER_TILE // ZROWS)
        def _(k):
            pltpu.sync_copy(zrow_v, u_sh.at[pl.ds(s * ROWS_PER_TILE + k * ZROWS, ZROWS)])

        pltpu.sync_copy(zrs_v, rs_sh.at[pl.ds(s * ROWS_PER_TILE, ROWS_PER_TILE)])

        pltpu.sync_copy(f1_hbm, f1_v)
        pltpu.sync_copy(f2_hbm, f2_v)

        plsc.subcore_barrier()

        @pl.loop(wid, NCHUNKS, step=NC * NS)
        def _(t):
            e0 = t * CHUNK
            pltpu.sync_copy(src_hbm.at[pl.ds(e0, CHUNK)], src_v.at[0])
            pltpu.sync_copy(dst_hbm.at[pl.ds(e0, CHUNK)], dst_v.at[0])
            pltpu.sync_copy(a_hbm.at[pl.ds(e0, CHUNK)], a_v)
            pltpu.async_copy(h_hbm.at[dst_v.at[0]], rows_v, sem).wait()

            for g in range(CHUNK // LANE):
                s16 = src_v[0, pl.ds(g * LANE, LANE)]
                d16 = dst_v[0, pl.ds(g * LANE, LANE)]
                a16 = a_v[pl.ds(g * LANE, LANE)]
                f1g = plsc.load_gather(f1_v, [s16])
                f2g = plsc.load_gather(f2_v, [d16])
                x = a16 * (f1g + f2g)
                att = 1.0 / (1.0 + jnp.exp(-x))
                w16 = jnp.exp(att)
                w_v[pl.ds(g * LANE, LANE)] = w16
                for j in range(LANE):
                    wj = jnp.take(w16, jnp.full((LANE,), j, jnp.int32))
                    r = g * LANE + j
                    for q in range(D // LANE):
                        rows_v[r, pl.ds(q * LANE, LANE)] = (
                            rows_v[r, pl.ds(q * LANE, LANE)] * wj)

            pltpu.sync_copy(w_v, rs_sh.at[src_v.at[0]], add=True)
            pltpu.sync_copy(rows_v, u_sh.at[src_v.at[0]], add=True)

        plsc.subcore_barrier()

        @pl.loop(0, ROWS_PER_TILE // ZROWS)
        def _(k):
            r0 = s * ROWS_PER_TILE + k * ZROWS
            pltpu.sync_copy(u_sh.at[pl.ds(r0, ZROWS)], u_out.at[c, pl.ds(r0, ZROWS)])

        pltpu.sync_copy(rs_sh.at[pl.ds(s * ROWS_PER_TILE, ROWS_PER_TILE)],
                        rs_out.at[c, 0, pl.ds(s * ROWS_PER_TILE, ROWS_PER_TILE)])

    return edge_kernel


_edge_kernel_64 = _make_edge_kernel(64)
_edge_kernel_32 = _make_edge_kernel(32)


def _dot(a, b):
    return lax.dot_general(a, b, (((1,), (0,)), ((), ())),
                           preferred_element_type=jnp.float32,
                           precision=lax.Precision.HIGHEST)


def _encode1_body(x_ref, w_ref, v0_ref, v1_ref, h_ref, f1_ref, f2_ref):
    h = _dot(x_ref[...], w_ref[...])
    h_ref[...] = h
    f1_ref[...] = _dot(h, v0_ref[...])
    f2_ref[...] = _dot(h, v1_ref[...])


def _encode1(X, W0, v0, v1):
    bm = 1000
    return pl.pallas_call(
        _encode1_body,
        grid=(N // bm,),
        in_specs=[
            pl.BlockSpec((bm, 128), lambda i: (i, 0)),
            pl.BlockSpec((128, 64), lambda i: (0, 0)),
            pl.BlockSpec((64, 1), lambda i: (0, 0)),
            pl.BlockSpec((64, 1), lambda i: (0, 0)),
        ],
        out_specs=[
            pl.BlockSpec((bm, 64), lambda i: (i, 0)),
            pl.BlockSpec((bm, 1), lambda i: (i, 0)),
            pl.BlockSpec((bm, 1), lambda i: (i, 0)),
        ],
        out_shape=[
            jax.ShapeDtypeStruct((N, 64), jnp.float32),
            jax.ShapeDtypeStruct((N, 1), jnp.float32),
            jax.ShapeDtypeStruct((N, 1), jnp.float32),
        ],
    )(X, W0, v0, v1)


def _combine_encode2_body(u0_ref, u1_ref, r0_ref, r1_ref, w_ref, v0_ref,
                          v1_ref, h_ref, f1_ref, f2_ref):
    rs = r0_ref[...] + r1_ref[...]
    rs = jnp.where(rs == 0.0, 1.0, rs)
    hin = (u0_ref[...] + u1_ref[...]) / rs
    h = _dot(hin, w_ref[...])
    h_ref[...] = h
    f1_ref[...] = _dot(h, v0_ref[...])
    f2_ref[...] = _dot(h, v1_ref[...])


def _combine_encode2(U0, U1, r0, r1, W1, v0, v1):
    bm = 1000
    return pl.pallas_call(
        _combine_encode2_body,
        grid=(N // bm,),
        in_specs=[
            pl.BlockSpec((bm, 64), lambda i: (i, 0)),
            pl.BlockSpec((bm, 64), lambda i: (i, 0)),
            pl.BlockSpec((bm, 1), lambda i: (i, 0)),
            pl.BlockSpec((bm, 1), lambda i: (i, 0)),
            pl.BlockSpec((64, 32), lambda i: (0, 0)),
            pl.BlockSpec((32, 1), lambda i: (0, 0)),
            pl.BlockSpec((32, 1), lambda i: (0, 0)),
        ],
        out_specs=[
            pl.BlockSpec((bm, 32), lambda i: (i, 0)),
            pl.BlockSpec((bm, 1), lambda i: (i, 0)),
            pl.BlockSpec((bm, 1), lambda i: (i, 0)),
        ],
        out_shape=[
            jax.ShapeDtypeStruct((N, 32), jnp.float32),
            jax.ShapeDtypeStruct((N, 1), jnp.float32),
            jax.ShapeDtypeStruct((N, 1), jnp.float32),
        ],
    )(U0, U1, r0, r1, W1, v0, v1)


def _combine_body(u0_ref, u1_ref, r0_ref, r1_ref, h_ref):
    rs = r0_ref[...] + r1_ref[...]
    rs = jnp.where(rs == 0.0, 1.0, rs)
    h_ref[...] = ((u0_ref[...] + u1_ref[...]) / rs).astype(jnp.bfloat16)


def _combine(U0, U1, r0, r1):
    bm = 1000
    return pl.pallas_call(
        _combine_body,
        grid=(N // bm,),
        in_specs=[
            pl.BlockSpec((bm, 32), lambda i: (i, 0)),
            pl.BlockSpec((bm, 32), lambda i: (i, 0)),
            pl.BlockSpec((bm, 1), lambda i: (i, 0)),
            pl.BlockSpec((bm, 1), lambda i: (i, 0)),
        ],
        out_specs=pl.BlockSpec((bm, 32), lambda i: (i, 0)),
        out_shape=jax.ShapeDtypeStruct((N, 32), jnp.bfloat16),
    )(U0, U1, r0, r1)


def _decode_body(a_ref, b_ref, o_ref):
    z = lax.dot_general(a_ref[...], b_ref[...], (((1,), (1,)), ((), ())),
                        preferred_element_type=jnp.float32)
    o_ref[...] = 0.5 * jnp.tanh(0.5 * z) + 0.5


def _decode(Hf):
    bm = 512
    g = pl.cdiv(N, bm)
    return pl.pallas_call(
        _decode_body,
        grid=(g, g),
        in_specs=[
            pl.BlockSpec((bm, 32), lambda i, j: (i, 0)),
            pl.BlockSpec((bm, 32), lambda i, j: (j, 0)),
        ],
        out_specs=pl.BlockSpec((bm, bm), lambda i, j: (i, j)),
        out_shape=jax.ShapeDtypeStruct((N, N), jnp.float32),
        compiler_params=pltpu.CompilerParams(
            dimension_semantics=("parallel", "parallel")),
    )(Hf, Hf)


def kernel(X, edge_index, A_values, W0, W1, v0_0, v1_0, v0_1, v1_1):
    src = edge_index[0]
    dst = edge_index[1]

    H1, f1a, f2a = _encode1(X, W0, v0_0, v1_0)
    U1, RS1 = _edge_kernel_64(src, dst, A_values, H1,
                              f1a.reshape(N), f2a.reshape(N))
    H2, f1b, f2b = _combine_encode2(
        U1[0, :N], U1[1, :N],
        RS1[0, 0, :N].reshape(N, 1), RS1[1, 0, :N].reshape(N, 1),
        W1, v0_1, v1_1)
    U2, RS2 = _edge_kernel_32(src, dst, A_values, H2,
                              f1b.reshape(N), f2b.reshape(N))
    Hf = _combine(U2[0, :N], U2[1, :N],
                  RS2[0, 0, :N].reshape(N, 1), RS2[1, 0, :N].reshape(N, 1))
    return _decode(Hf)


# speedup vs baseline: 21.5204x; 1.3531x over previous
import functools

import jax
import jax.numpy as jnp
from jax import lax
from jax.experimental import pallas as pl
from jax.experimental.pallas import tpu as pltpu
from jax.experimental.pallas import tpu_sc as plsc

N = 10000
E = 320000
NC = 2
NS = 16
LANE = 16
CHUNK = 128
NCHUNKS = E // CHUNK
NPAD = 10240
ROWS_PER_TILE = NPAD // NS
ZROWS = 128

_SC_MESH = plsc.VectorSubcoreMesh(core_axis_name="c", subcore_axis_name="s")
_SC_PARAMS = pltpu.CompilerParams(needs_layout_passes=False,
                                  use_tc_tiling_on_sc=False)


def _make_edge_kernel(D):
    @functools.partial(
        pl.kernel,
        out_type=[
            jax.ShapeDtypeStruct((NC, NPAD, D), jnp.float32),
            jax.ShapeDtypeStruct((NC, 1, NPAD), jnp.float32),
        ],
        mesh=_SC_MESH,
        compiler_params=_SC_PARAMS,
        scratch_types=[
            pltpu.VMEM((N,), jnp.float32),
            pltpu.VMEM((N,), jnp.float32),
            pltpu.VMEM((2, 2, CHUNK), jnp.int32),
            pltpu.VMEM((2, CHUNK), jnp.float32),
            pltpu.VMEM((2, CHUNK), jnp.float32),
            pltpu.VMEM((2, CHUNK, D), jnp.float32),
            pltpu.VMEM((ZROWS, D), jnp.float32),
            pltpu.VMEM((ROWS_PER_TILE,), jnp.float32),
            pltpu.VMEM_SHARED((NPAD, D), jnp.float32),
            pltpu.VMEM_SHARED((NPAD,), jnp.float32),
            pltpu.SemaphoreType.DMA((2,)),
            pltpu.SemaphoreType.DMA((2,)),
        ],
    )
    def edge_kernel(ei_hbm, a_hbm, h_hbm, f1_hbm, f2_hbm,
                    u_out, rs_out,
                    f1_v, f2_v, idx_v, a_v, w_v, rows_v,
                    zrow_v, zrs_v, u_sh, rs_sh, sem_i, sem_g):
        c = lax.axis_index("c")
        s = lax.axis_index("s")
        wid = c * NS + s
        nct = jnp.where(wid < NCHUNKS % (NC * NS), NCHUNKS // (NC * NS) + 1,
                        NCHUNKS // (NC * NS))

        zero16 = jnp.zeros((LANE,), jnp.float32)

        def issue_idx(k, b):
            e0 = (wid + (NC * NS) * k) * CHUNK
            pltpu.async_copy(ei_hbm.at[:, pl.ds(e0, CHUNK)], idx_v.at[b],
                             sem_i.at[b])
            pltpu.async_copy(a_hbm.at[pl.ds(e0, CHUNK)], a_v.at[b],
                             sem_i.at[b])

        def wait_idx(b):
            pltpu.make_async_copy(ei_hbm.at[:, pl.ds(0, CHUNK)], idx_v.at[b],
                                  sem_i.at[b]).wait()
            pltpu.make_async_copy(a_hbm.at[pl.ds(0, CHUNK)], a_v.at[b],
                                  sem_i.at[b]).wait()

        def issue_gather(b):
            pltpu.async_copy(h_hbm.at[idx_v.at[b, 1]], rows_v.at[b],
                             sem_g.at[b])

        def wait_gather(b):
            pltpu.make_async_copy(h_hbm.at[idx_v.at[b, 1]], rows_v.at[b],
                                  sem_g.at[b]).wait()

        issue_idx(0, 0)
        issue_idx(1, 1)

        @pl.loop(0, ZROWS)
        def _(i):
            @pl.loop(0, D, step=LANE)
            def _(q):
                zrow_v[i, pl.ds(q, LANE)] = zero16

        @pl.loop(0, ROWS_PER_TILE, step=LANE)
        def _(q):
            zrs_v[pl.ds(q, LANE)] = zero16

        @pl.loop(0, ROWS_PER_TILE // ZROWS)
        def _(k):
            pltpu.sync_copy(zrow_v, u_sh.at[pl.ds(s * ROWS_PER_TILE + k * ZROWS, ZROWS)])

        pltpu.sync_copy(zrs_v, rs_sh.at[pl.ds(s * ROWS_PER_TILE, ROWS_PER_TILE)])

        pltpu.sync_copy(f1_hbm, f1_v)
        pltpu.sync_copy(f2_hbm, f2_v)

        wait_idx(0)
        issue_gather(0)

        plsc.subcore_barrier()

        @pl.loop(0, NCHUNKS // (NC * NS) + 1)
        def _(k):
            @pl.when(k < nct)
            def _():
                b = k & 1

                @pl.when(k + 1 < nct)
                def _():
                    wait_idx(1 - b)
                    issue_gather(1 - b)

                for g in range(CHUNK // LANE):
                    s16 = idx_v[b, 0, pl.ds(g * LANE, LANE)]
                    d16 = idx_v[b, 1, pl.ds(g * LANE, LANE)]
                    a16 = a_v[b, pl.ds(g * LANE, LANE)]
                    f1g = plsc.load_gather(f1_v, [s16])
                    f2g = plsc.load_gather(f2_v, [d16])
                    x = a16 * (f1g + f2g)
                    att = 1.0 / (1.0 + jnp.exp(-x))
                    w16 = jnp.exp(att)
                    w_v[b, pl.ds(g * LANE, LANE)] = w16

                pltpu.sync_copy(w_v.at[b], rs_sh.at[idx_v.at[b, 0]], add=True)

                wait_gather(b)

                for g in range(CHUNK // LANE):
                    w16 = w_v[b, pl.ds(g * LANE, LANE)]
                    for j in range(LANE):
                        wj = jnp.take(w16, jnp.full((LANE,), j, jnp.int32))
                        r = g * LANE + j
                        for q in range(D // LANE):
                            rows_v[b, r, pl.ds(q * LANE, LANE)] = (
                                rows_v[b, r, pl.ds(q * LANE, LANE)] * wj)

                pltpu.sync_copy(rows_v.at[b], u_sh.at[idx_v.at[b, 0]], add=True)

                @pl.when(k + 2 < nct)
                def _():
                    issue_idx(k + 2, b)

        plsc.subcore_barrier()

        @pl.loop(0, ROWS_PER_TILE // ZROWS)
        def _(k):
            r0 = s * ROWS_PER_TILE + k * ZROWS
            pltpu.sync_copy(u_sh.at[pl.ds(r0, ZROWS)], u_out.at[c, pl.ds(r0, ZROWS)])

        pltpu.sync_copy(rs_sh.at[pl.ds(s * ROWS_PER_TILE, ROWS_PER_TILE)],
                        rs_out.at[c, 0, pl.ds(s * ROWS_PER_TILE, ROWS_PER_TILE)])

    return edge_kernel


_edge_kernel_64 = _make_edge_kernel(64)
_edge_kernel_32 = _make_edge_kernel(32)


def _dot(a, b):
    return lax.dot_general(a, b, (((1,), (0,)), ((), ())),
                           preferred_element_type=jnp.float32,
                           precision=lax.Precision.HIGHEST)


def _encode1_body(x_ref, w_ref, v0_ref, v1_ref, h_ref, f1_ref, f2_ref):
    h = _dot(x_ref[...], w_ref[...])
    h_ref[...] = h
    f1_ref[...] = _dot(h, v0_ref[...])
    f2_ref[...] = _dot(h, v1_ref[...])


def _encode1(X, W0, v0, v1):
    bm = 1000
    return pl.pallas_call(
        _encode1_body,
        grid=(N // bm,),
        in_specs=[
            pl.BlockSpec((bm, 128), lambda i: (i, 0)),
            pl.BlockSpec((128, 64), lambda i: (0, 0)),
            pl.BlockSpec((64, 1), lambda i: (0, 0)),
            pl.BlockSpec((64, 1), lambda i: (0, 0)),
        ],
        out_specs=[
            pl.BlockSpec((bm, 64), lambda i: (i, 0)),
            pl.BlockSpec((bm, 1), lambda i: (i, 0)),
            pl.BlockSpec((bm, 1), lambda i: (i, 0)),
        ],
        out_shape=[
            jax.ShapeDtypeStruct((N, 64), jnp.float32),
            jax.ShapeDtypeStruct((N, 1), jnp.float32),
            jax.ShapeDtypeStruct((N, 1), jnp.float32),
        ],
    )(X, W0, v0, v1)


def _combine_encode2_body(u0_ref, u1_ref, r0_ref, r1_ref, w_ref, v0_ref,
                          v1_ref, h_ref, f1_ref, f2_ref):
    rs = r0_ref[...] + r1_ref[...]
    rs = jnp.where(rs == 0.0, 1.0, rs)
    hin = (u0_ref[...] + u1_ref[...]) / rs
    h = _dot(hin, w_ref[...])
    h_ref[...] = h
    f1_ref[...] = _dot(h, v0_ref[...])
    f2_ref[...] = _dot(h, v1_ref[...])


def _combine_encode2(U0, U1, r0, r1, W1, v0, v1):
    bm = 1000
    return pl.pallas_call(
        _combine_encode2_body,
        grid=(N // bm,),
        in_specs=[
            pl.BlockSpec((bm, 64), lambda i: (i, 0)),
            pl.BlockSpec((bm, 64), lambda i: (i, 0)),
            pl.BlockSpec((bm, 1), lambda i: (i, 0)),
            pl.BlockSpec((bm, 1), lambda i: (i, 0)),
            pl.BlockSpec((64, 32), lambda i: (0, 0)),
            pl.BlockSpec((32, 1), lambda i: (0, 0)),
            pl.BlockSpec((32, 1), lambda i: (0, 0)),
        ],
        out_specs=[
            pl.BlockSpec((bm, 32), lambda i: (i, 0)),
            pl.BlockSpec((bm, 1), lambda i: (i, 0)),
            pl.BlockSpec((bm, 1), lambda i: (i, 0)),
        ],
        out_shape=[
            jax.ShapeDtypeStruct((N, 32), jnp.float32),
            jax.ShapeDtypeStruct((N, 1), jnp.float32),
            jax.ShapeDtypeStruct((N, 1), jnp.float32),
        ],
    )(U0, U1, r0, r1, W1, v0, v1)


def _combine_body(u0_ref, u1_ref, r0_ref, r1_ref, h_ref):
    rs = r0_ref[...] + r1_ref[...]
    rs = jnp.where(rs == 0.0, 1.0, rs)
    h_ref[...] = ((u0_ref[...] + u1_ref[...]) / rs).astype(jnp.bfloat16)


def _combine(U0, U1, r0, r1):
    bm = 1000
    return pl.pallas_call(
        _combine_body,
        grid=(N // bm,),
        in_specs=[
            pl.BlockSpec((bm, 32), lambda i: (i, 0)),
            pl.BlockSpec((bm, 32), lambda i: (i, 0)),
            pl.BlockSpec((bm, 1), lambda i: (i, 0)),
            pl.BlockSpec((bm, 1), lambda i: (i, 0)),
        ],
        out_specs=pl.BlockSpec((bm, 32), lambda i: (i, 0)),
        out_shape=jax.ShapeDtypeStruct((N, 32), jnp.bfloat16),
    )(U0, U1, r0, r1)


def _decode_body(a_ref, b_ref, o_ref):
    z = lax.dot_general(a_ref[...], b_ref[...], (((1,), (1,)), ((), ())),
                        preferred_element_type=jnp.float32)
    o_ref[...] = 0.5 * jnp.tanh(0.5 * z) + 0.5


def _decode(Hf):
    bm = 512
    g = pl.cdiv(N, bm)
    return pl.pallas_call(
        _decode_body,
        grid=(g, g),
        in_specs=[
            pl.BlockSpec((bm, 32), lambda i, j: (i, 0)),
            pl.BlockSpec((bm, 32), lambda i, j: (j, 0)),
        ],
        out_specs=pl.BlockSpec((bm, bm), lambda i, j: (i, j)),
        out_shape=jax.ShapeDtypeStruct((N, N), jnp.float32),
        compiler_params=pltpu.CompilerParams(
            dimension_semantics=("parallel", "parallel")),
    )(Hf, Hf)


def kernel(X, edge_index, A_values, W0, W1, v0_0, v1_0, v0_1, v1_1):
    H1, f1a, f2a = _encode1(X, W0, v0_0, v1_0)
    U1, RS1 = _edge_kernel_64(edge_index, A_values, H1,
                              f1a.reshape(N), f2a.reshape(N))
    H2, f1b, f2b = _combine_encode2(
        U1[0, :N], U1[1, :N],
        RS1[0, 0, :N].reshape(N, 1), RS1[1, 0, :N].reshape(N, 1),
        W1, v0_1, v1_1)
    U2, RS2 = _edge_kernel_32(edge_index, A_values, H2,
                              f1b.reshape(N), f2b.reshape(N))
    Hf = _combine(U2[0, :N], U2[1, :N],
                  RS2[0, 0, :N].reshape(N, 1), RS2[1, 0, :N].reshape(N, 1))
    return _decode(Hf)


# speedup vs baseline: 27.4081x; 1.2736x over previous
import functools

import jax
import jax.numpy as jnp
from jax import lax
from jax.experimental import pallas as pl
from jax.experimental.pallas import tpu as pltpu
from jax.experimental.pallas import tpu_sc as plsc

N = 10000
E = 320000
NC = 2
NS = 16
LANE = 16
CHUNK = 128
NCHUNKS = E // CHUNK
NPAD = 10240
ROWS_PER_TILE = NPAD // NS
ZROWS = 128

_SC_MESH = plsc.VectorSubcoreMesh(core_axis_name="c", subcore_axis_name="s")
_SC_PARAMS = pltpu.CompilerParams(needs_layout_passes=False,
                                  use_tc_tiling_on_sc=False)


def _make_edge_kernel(D):
    @functools.partial(
        pl.kernel,
        out_type=[
            jax.ShapeDtypeStruct((NC, NPAD, D), jnp.float32),
            jax.ShapeDtypeStruct((NC, 1, NPAD), jnp.float32),
        ],
        mesh=_SC_MESH,
        compiler_params=_SC_PARAMS,
        scratch_types=[
            pltpu.VMEM((N,), jnp.float32),
            pltpu.VMEM((N,), jnp.float32),
            pltpu.VMEM((2, 2, CHUNK), jnp.int32),
            pltpu.VMEM((2, CHUNK), jnp.float32),
            pltpu.VMEM((2, CHUNK), jnp.float32),
            pltpu.VMEM((2, CHUNK, D), jnp.float32),
            pltpu.VMEM((ZROWS, D), jnp.float32),
            pltpu.VMEM((ROWS_PER_TILE,), jnp.float32),
            pltpu.VMEM_SHARED((NPAD, D), jnp.float32),
            pltpu.VMEM_SHARED((NPAD,), jnp.float32),
            pltpu.SemaphoreType.DMA((2,)),
            pltpu.SemaphoreType.DMA((2,)),
        ],
    )
    def edge_kernel(ei_hbm, a_hbm, h_hbm, f1_hbm, f2_hbm,
                    u_out, rs_out,
                    f1_v, f2_v, idx_v, a_v, w_v, rows_v,
                    zrow_v, zrs_v, u_sh, rs_sh, sem_i, sem_g):
        c = lax.axis_index("c")
        s = lax.axis_index("s")
        wid = c * NS + s
        nct = jnp.where(wid < NCHUNKS % (NC * NS), NCHUNKS // (NC * NS) + 1,
                        NCHUNKS // (NC * NS))

        zero16 = jnp.zeros((LANE,), jnp.float32)

        def issue_idx(k, b):
            e0 = (wid + (NC * NS) * k) * CHUNK
            pltpu.async_copy(ei_hbm.at[:, pl.ds(e0, CHUNK)], idx_v.at[b],
                             sem_i.at[b])
            pltpu.async_copy(a_hbm.at[pl.ds(e0, CHUNK)], a_v.at[b],
                             sem_i.at[b])

        def wait_idx(b):
            pltpu.make_async_copy(ei_hbm.at[:, pl.ds(0, CHUNK)], idx_v.at[b],
                                  sem_i.at[b]).wait()
            pltpu.make_async_copy(a_hbm.at[pl.ds(0, CHUNK)], a_v.at[b],
                                  sem_i.at[b]).wait()

        def issue_gather(b):
            pltpu.async_copy(h_hbm.at[idx_v.at[b, 1]], rows_v.at[b],
                             sem_g.at[b])

        def wait_gather(b):
            pltpu.make_async_copy(h_hbm.at[idx_v.at[b, 1]], rows_v.at[b],
                                  sem_g.at[b]).wait()

        issue_idx(0, 0)
        issue_idx(1, 1)

        @pl.loop(0, ZROWS)
        def _(i):
            @pl.loop(0, D, step=LANE)
            def _(q):
                zrow_v[i, pl.ds(q, LANE)] = zero16

        @pl.loop(0, ROWS_PER_TILE, step=LANE)
        def _(q):
            zrs_v[pl.ds(q, LANE)] = zero16

        @pl.loop(0, ROWS_PER_TILE // ZROWS)
        def _(k):
            pltpu.sync_copy(zrow_v, u_sh.at[pl.ds(s * ROWS_PER_TILE + k * ZROWS, ZROWS)])

        pltpu.sync_copy(zrs_v, rs_sh.at[pl.ds(s * ROWS_PER_TILE, ROWS_PER_TILE)])

        pltpu.sync_copy(f1_hbm, f1_v)
        pltpu.sync_copy(f2_hbm, f2_v)

        wait_idx(0)
        issue_gather(0)

        plsc.subcore_barrier()

        @pl.loop(0, NCHUNKS // (NC * NS) + 1)
        def _(k):
            @pl.when(k < nct)
            def _():
                b = k & 1

                @pl.when(k + 1 < nct)
                def _():
                    wait_idx(1 - b)
                    issue_gather(1 - b)

                for g in range(CHUNK // LANE):
                    s16 = idx_v[b, 0, pl.ds(g * LANE, LANE)]
                    d16 = idx_v[b, 1, pl.ds(g * LANE, LANE)]
                    a16 = a_v[b, pl.ds(g * LANE, LANE)]
                    f1g = plsc.load_gather(f1_v, [s16])
                    f2g = plsc.load_gather(f2_v, [d16])
                    x = a16 * (f1g + f2g)
                    att = 1.0 / (1.0 + jnp.exp(-x))
                    w16 = jnp.exp(att)
                    w_v[b, pl.ds(g * LANE, LANE)] = w16

                pltpu.sync_copy(w_v.at[b], rs_sh.at[idx_v.at[b, 0]], add=True)

                wait_gather(b)

                for g in range(CHUNK // LANE):
                    w16 = w_v[b, pl.ds(g * LANE, LANE)]
                    for j in range(LANE):
                        wj = jnp.take(w16, jnp.full((LANE,), j, jnp.int32))
                        r = g * LANE + j
                        for q in range(D // LANE):
                            rows_v[b, r, pl.ds(q * LANE, LANE)] = (
                                rows_v[b, r, pl.ds(q * LANE, LANE)] * wj)

                pltpu.sync_copy(rows_v.at[b], u_sh.at[idx_v.at[b, 0]], add=True)

                @pl.when(k + 2 < nct)
                def _():
                    issue_idx(k + 2, b)

        plsc.subcore_barrier()

        @pl.loop(0, ROWS_PER_TILE // ZROWS)
        def _(k):
            r0 = s * ROWS_PER_TILE + k * ZROWS
            pltpu.sync_copy(u_sh.at[pl.ds(r0, ZROWS)], u_out.at[c, pl.ds(r0, ZROWS)])

        pltpu.sync_copy(rs_sh.at[pl.ds(s * ROWS_PER_TILE, ROWS_PER_TILE)],
                        rs_out.at[c, 0, pl.ds(s * ROWS_PER_TILE, ROWS_PER_TILE)])

    return edge_kernel


_edge_kernel_64 = _make_edge_kernel(64)
_edge_kernel_32 = _make_edge_kernel(32)


def _dot(a, b):
    return lax.dot_general(a, b, (((1,), (0,)), ((), ())),
                           preferred_element_type=jnp.float32)


def _encode1_body(x_ref, w_ref, v0_ref, v1_ref, h_ref, f1_ref, f2_ref):
    h = _dot(x_ref[...], w_ref[...])
    h_ref[...] = h
    f1_ref[...] = _dot(h, v0_ref[...])
    f2_ref[...] = _dot(h, v1_ref[...])


def _encode1(X, W0, v0, v1):
    bm = 1000
    return pl.pallas_call(
        _encode1_body,
        grid=(N // bm,),
        in_specs=[
            pl.BlockSpec((bm, 128), lambda i: (i, 0)),
            pl.BlockSpec((128, 64), lambda i: (0, 0)),
            pl.BlockSpec((64, 1), lambda i: (0, 0)),
            pl.BlockSpec((64, 1), lambda i: (0, 0)),
        ],
        out_specs=[
            pl.BlockSpec((bm, 64), lambda i: (i, 0)),
            pl.BlockSpec((bm, 1), lambda i: (i, 0)),
            pl.BlockSpec((bm, 1), lambda i: (i, 0)),
        ],
        out_shape=[
            jax.ShapeDtypeStruct((N, 64), jnp.float32),
            jax.ShapeDtypeStruct((N, 1), jnp.float32),
            jax.ShapeDtypeStruct((N, 1), jnp.float32),
        ],
    )(X, W0, v0, v1)


def _combine_encode2_body(u0_ref, u1_ref, r0_ref, r1_ref, w_ref, v0_ref,
                          v1_ref, h_ref, f1_ref, f2_ref):
    rs = r0_ref[...] + r1_ref[...]
    rs = jnp.where(rs == 0.0, 1.0, rs)
    hin = (u0_ref[...] + u1_ref[...]) / rs
    h = _dot(hin, w_ref[...])
    h_ref[...] = h
    f1_ref[...] = _dot(h, v0_ref[...])
    f2_ref[...] = _dot(h, v1_ref[...])


def _combine_encode2(U0, U1, r0, r1, W1, v0, v1):
    bm = 1000
    return pl.pallas_call(
        _combine_encode2_body,
        grid=(N // bm,),
        in_specs=[
            pl.BlockSpec((bm, 64), lambda i: (i, 0)),
            pl.BlockSpec((bm, 64), lambda i: (i, 0)),
            pl.BlockSpec((bm, 1), lambda i: (i, 0)),
            pl.BlockSpec((bm, 1), lambda i: (i, 0)),
            pl.BlockSpec((64, 32), lambda i: (0, 0)),
            pl.BlockSpec((32, 1), lambda i: (0, 0)),
            pl.BlockSpec((32, 1), lambda i: (0, 0)),
        ],
        out_specs=[
            pl.BlockSpec((bm, 32), lambda i: (i, 0)),
            pl.BlockSpec((bm, 1), lambda i: (i, 0)),
            pl.BlockSpec((bm, 1), lambda i: (i, 0)),
        ],
        out_shape=[
            jax.ShapeDtypeStruct((N, 32), jnp.float32),
            jax.ShapeDtypeStruct((N, 1), jnp.float32),
            jax.ShapeDtypeStruct((N, 1), jnp.float32),
        ],
    )(U0, U1, r0, r1, W1, v0, v1)


def _combine_body(u0_ref, u1_ref, r0_ref, r1_ref, h_ref):
    rs = r0_ref[...] + r1_ref[...]
    rs = jnp.where(rs == 0.0, 1.0, rs)
    h_ref[...] = ((u0_ref[...] + u1_ref[...]) / rs).astype(jnp.bfloat16)


def _combine(U0, U1, r0, r1):
    bm = 1000
    return pl.pallas_call(
        _combine_body,
        grid=(N // bm,),
        in_specs=[
            pl.BlockSpec((bm, 32), lambda i: (i, 0)),
            pl.BlockSpec((bm, 32), lambda i: (i, 0)),
            pl.BlockSpec((bm, 1), lambda i: (i, 0)),
            pl.BlockSpec((bm, 1), lambda i: (i, 0)),
        ],
        out_specs=pl.BlockSpec((bm, 32), lambda i: (i, 0)),
        out_shape=jax.ShapeDtypeStruct((N, 32), jnp.bfloat16),
    )(U0, U1, r0, r1)


def _decode_body(a_ref, b_ref, o_ref):
    z = lax.dot_general(a_ref[...], b_ref[...], (((1,), (1,)), ((), ())),
                        preferred_element_type=jnp.float32)
    o_ref[...] = 0.5 * jnp.tanh(0.5 * z) + 0.5


def _decode(Hf):
    bm, bn = 512, 1024
    return pl.pallas_call(
        _decode_body,
        grid=(pl.cdiv(N, bm), pl.cdiv(N, bn)),
        in_specs=[
            pl.BlockSpec((bm, 32), lambda i, j: (i, 0)),
            pl.BlockSpec((bn, 32), lambda i, j: (j, 0)),
        ],
        out_specs=pl.BlockSpec((bm, bn), lambda i, j: (i, j)),
        out_shape=jax.ShapeDtypeStruct((N, N), jnp.float32),
        compiler_params=pltpu.CompilerParams(
            dimension_semantics=("parallel", "parallel")),
    )(Hf, Hf)


def kernel(X, edge_index, A_values, W0, W1, v0_0, v1_0, v0_1, v1_1):
    H1, f1a, f2a = _encode1(X, W0, v0_0, v1_0)
    U1, RS1 = _edge_kernel_64(edge_index, A_values, H1,
                              f1a.reshape(N), f2a.reshape(N))
    H2, f1b, f2b = _combine_encode2(
        U1[0, :N], U1[1, :N],
        RS1[0, 0, :N].reshape(N, 1), RS1[1, 0, :N].reshape(N, 1),
        W1, v0_1, v1_1)
    U2, RS2 = _edge_kernel_32(edge_index, A_values, H2,
                              f1b.reshape(N), f2b.reshape(N))
    Hf = _combine(U2[0, :N], U2[1, :N],
                  RS2[0, 0, :N].reshape(N, 1), RS2[1, 0, :N].reshape(N, 1))
    return _decode(Hf)


# speedup vs baseline: 35.9089x; 1.3102x over previous
import functools

import jax
import jax.numpy as jnp
from jax import lax
from jax.experimental import pallas as pl
from jax.experimental.pallas import tpu as pltpu
from jax.experimental.pallas import tpu_sc as plsc

N = 10000
E = 320000
NC = 2
NS = 16
NW = NC * NS
LANE = 16
CHUNK = 128
NCHUNKS = E // CHUNK
NPAD = 10240
ROWS_PER_TILE = NPAD // NS
ZROWS = 128

_SC_MESH = plsc.VectorSubcoreMesh(core_axis_name="c", subcore_axis_name="s")
_SC_PARAMS = pltpu.CompilerParams(needs_layout_passes=False,
                                  use_tc_tiling_on_sc=False)


def _make_edge_kernel(D):
    @functools.partial(
        pl.kernel,
        out_type=[
            jax.ShapeDtypeStruct((NC, NPAD, D), jnp.float32),
            jax.ShapeDtypeStruct((NC, 1, NPAD), jnp.float32),
        ],
        mesh=_SC_MESH,
        compiler_params=_SC_PARAMS,
        scratch_types=[
            pltpu.VMEM((2, N), jnp.float32),
            pltpu.VMEM((2, 2, CHUNK), jnp.int32),
            pltpu.VMEM((2, 1, CHUNK), jnp.int32),
            pltpu.VMEM((2, CHUNK), jnp.float32),
            pltpu.VMEM((2, CHUNK), jnp.float32),
            pltpu.VMEM((2, CHUNK, D), jnp.float32),
            pltpu.VMEM((ZROWS, D), jnp.float32),
            pltpu.VMEM((ROWS_PER_TILE,), jnp.float32),
            pltpu.VMEM_SHARED((NPAD, D), jnp.float32),
            pltpu.VMEM_SHARED((NPAD,), jnp.float32),
            pltpu.SemaphoreType.DMA((2,)),
            pltpu.SemaphoreType.DMA((2,)),
            pltpu.SemaphoreType.DMA((2,)),
        ],
    )
    def edge_kernel(ei_hbm, a_hbm, h_hbm, f_hbm,
                    u_out, rs_out,
                    f_v, idx_v, sidx_v, a_v, w_v, rows_v,
                    zrow_v, zrs_v, u_sh, rs_sh, sem_i, sem_g, sem_s):
        c = lax.axis_index("c")
        s = lax.axis_index("s")
        wid = c * NS + s
        nct = jnp.where(wid < NCHUNKS % NW, NCHUNKS // NW + 1, NCHUNKS // NW)

        zero16 = jnp.zeros((LANE,), jnp.float32)
        zrow16 = jnp.zeros((LANE,), jnp.int32)
        onerow16 = jnp.full((LANE,), 1, jnp.int32)

        def issue_idx(k, b):
            e0 = (wid + NW * k) * CHUNK
            pltpu.async_copy(ei_hbm.at[:, pl.ds(e0, CHUNK)], idx_v.at[b],
                             sem_i.at[b])
            pltpu.async_copy(a_hbm.at[pl.ds(e0, CHUNK)], a_v.at[b],
                             sem_i.at[b])

        def wait_idx(b):
            pltpu.make_async_copy(ei_hbm.at[:, pl.ds(0, CHUNK)], idx_v.at[b],
                                  sem_i.at[b]).wait()
            pltpu.make_async_copy(a_hbm.at[pl.ds(0, CHUNK)], a_v.at[b],
                                  sem_i.at[b]).wait()

        def issue_gather(b):
            pltpu.async_copy(h_hbm.at[idx_v.at[b, 1]], rows_v.at[b],
                             sem_g.at[b])

        def wait_gather(b):
            pltpu.make_async_copy(h_hbm.at[idx_v.at[b, 1]], rows_v.at[b],
                                  sem_g.at[b]).wait()

        def issue_scatter_rows(b):
            pltpu.async_copy(rows_v.at[b], u_sh.at[sidx_v.at[b, 0]],
                             sem_s.at[b], add=True)

        def wait_scatter_rows(b):
            pltpu.make_async_copy(rows_v.at[b], u_sh.at[sidx_v.at[b, 0]],
                                  sem_s.at[b]).wait()

        issue_idx(0, 0)
        issue_idx(1, 1)

        @pl.loop(0, ZROWS)
        def _(i):
            @pl.loop(0, D, step=LANE)
            def _(q):
                zrow_v[i, pl.ds(q, LANE)] = zero16

        @pl.loop(0, ROWS_PER_TILE, step=LANE)
        def _(q):
            zrs_v[pl.ds(q, LANE)] = zero16

        @pl.loop(0, ROWS_PER_TILE // ZROWS)
        def _(k):
            pltpu.sync_copy(zrow_v, u_sh.at[pl.ds(s * ROWS_PER_TILE + k * ZROWS, ZROWS)])

        pltpu.sync_copy(zrs_v, rs_sh.at[pl.ds(s * ROWS_PER_TILE, ROWS_PER_TILE)])

        pltpu.sync_copy(f_hbm, f_v)

        wait_idx(0)
        issue_gather(0)

        plsc.subcore_barrier()

        @pl.loop(0, NCHUNKS // NW + 1)
        def _(k):
            @pl.when(k < nct)
            def _():
                b = k & 1

                @pl.when(k + 1 < nct)
                def _():
                    wait_idx(1 - b)

                    @pl.when(k >= 1)
                    def _():
                        wait_scatter_rows(1 - b)

                    issue_gather(1 - b)

                for g in range(CHUNK // LANE):
                    s16 = idx_v[b, 0, pl.ds(g * LANE, LANE)]
                    d16 = idx_v[b, 1, pl.ds(g * LANE, LANE)]
                    a16 = a_v[b, pl.ds(g * LANE, LANE)]
                    sidx_v[b, 0, pl.ds(g * LANE, LANE)] = s16
                    f1g = plsc.load_gather(f_v, [zrow16, s16])
                    f2g = plsc.load_gather(f_v, [onerow16, d16])
                    x = a16 * (f1g + f2g)
                    att = 1.0 / (1.0 + jnp.exp(-x))
                    w16 = jnp.exp(att)
                    w_v[b, pl.ds(g * LANE, LANE)] = w16

                pltpu.sync_copy(w_v.at[b], rs_sh.at[sidx_v.at[b, 0]], add=True)

                wait_gather(b)

                for g in range(CHUNK // LANE):
                    w16 = w_v[b, pl.ds(g * LANE, LANE)]
                    for j in range(LANE):
                        wj = jnp.take(w16, jnp.full((LANE,), j, jnp.int32))
                        r = g * LANE + j
                        for q in range(D // LANE):
                            rows_v[b, r, pl.ds(q * LANE, LANE)] = (
                                rows_v[b, r, pl.ds(q * LANE, LANE)] * wj)

                issue_scatter_rows(b)

                @pl.when(k + 2 < nct)
                def _():
                    issue_idx(k + 2, b)

        wait_scatter_rows((nct - 1) & 1)

        @pl.when(nct >= 2)
        def _():
            wait_scatter_rows(nct & 1)

        plsc.subcore_barrier()

        @pl.loop(0, ROWS_PER_TILE // ZROWS)
        def _(k):
            r0 = s * ROWS_PER_TILE + k * ZROWS
            pltpu.sync_copy(u_sh.at[pl.ds(r0, ZROWS)], u_out.at[c, pl.ds(r0, ZROWS)])

        pltpu.sync_copy(rs_sh.at[pl.ds(s * ROWS_PER_TILE, ROWS_PER_TILE)],
                        rs_out.at[c, 0, pl.ds(s * ROWS_PER_TILE, ROWS_PER_TILE)])

    return edge_kernel


_edge_kernel_64 = _make_edge_kernel(64)
_edge_kernel_32 = _make_edge_kernel(32)


def _dot(a, b):
    return lax.dot_general(a, b, (((1,), (0,)), ((), ())),
                           preferred_element_type=jnp.float32)


def _frow(h, v):
    return lax.dot_general(v, h, (((0,), (1,)), ((), ())),
                           preferred_element_type=jnp.float32)


def _encode1_body(x_ref, w_ref, v0_ref, v1_ref, h_ref, f_ref):
    h = _dot(x_ref[...], w_ref[...])
    h_ref[...] = h
    f_ref[...] = jnp.concatenate(
        [_frow(h, v0_ref[...]), _frow(h, v1_ref[...])], axis=0)


def _encode1(X, W0, v0, v1):
    bm = 1024
    return pl.pallas_call(
        _encode1_body,
        grid=(pl.cdiv(N, bm),),
        in_specs=[
            pl.BlockSpec((bm, 128), lambda i: (i, 0)),
            pl.BlockSpec((128, 64), lambda i: (0, 0)),
            pl.BlockSpec((64, 1), lambda i: (0, 0)),
            pl.BlockSpec((64, 1), lambda i: (0, 0)),
        ],
        out_specs=[
            pl.BlockSpec((bm, 64), lambda i: (i, 0)),
            pl.BlockSpec((2, bm), lambda i: (0, i)),
        ],
        out_shape=[
            jax.ShapeDtypeStruct((N, 64), jnp.float32),
            jax.ShapeDtypeStruct((2, N), jnp.float32),
        ],
    )(X, W0, v0, v1)


def _combine_encode2_body(u_ref, r_ref, w_ref, v0_ref, v1_ref, h_ref, f_ref):
    rs = r_ref[0:1, :] + r_ref[1:2, :]
    rs = jnp.where(rs == 0.0, 1.0, rs)
    rs = jnp.transpose(rs, (1, 0))
    hin = (u_ref[0] + u_ref[1]) / rs
    h = _dot(hin, w_ref[...])
    h_ref[...] = h
    f_ref[...] = jnp.concatenate(
        [_frow(h, v0_ref[...]), _frow(h, v1_ref[...])], axis=0)


def _combine_encode2(U, R, W1, v0, v1):
    bm = 1024
    return pl.pallas_call(
        _combine_encode2_body,
        grid=(NPAD // bm,),
        in_specs=[
            pl.BlockSpec((2, bm, 64), lambda i: (0, i, 0)),
            pl.BlockSpec((2, bm), lambda i: (0, i)),
            pl.BlockSpec((64, 32), lambda i: (0, 0)),
            pl.BlockSpec((32, 1), lambda i: (0, 0)),
            pl.BlockSpec((32, 1), lambda i: (0, 0)),
        ],
        out_specs=[
            pl.BlockSpec((bm, 32), lambda i: (i, 0)),
            pl.BlockSpec((2, bm), lambda i: (0, i)),
        ],
        out_shape=[
            jax.ShapeDtypeStruct((N, 32), jnp.float32),
            jax.ShapeDtypeStruct((2, N), jnp.float32),
        ],
    )(U, R, W1, v0, v1)


def _combine_body(u_ref, r_ref, h_ref):
    rs = r_ref[0:1, :] + r_ref[1:2, :]
    rs = jnp.where(rs == 0.0, 1.0, rs)
    rs = jnp.transpose(rs, (1, 0))
    h_ref[...] = ((u_ref[0] + u_ref[1]) / rs).astype(jnp.bfloat16)


def _combine(U, R):
    bm = 1024
    return pl.pallas_call(
        _combine_body,
        grid=(NPAD // bm,),
        in_specs=[
            pl.BlockSpec((2, bm, 32), lambda i: (0, i, 0)),
            pl.BlockSpec((2, bm), lambda i: (0, i)),
        ],
        out_specs=pl.BlockSpec((bm, 32), lambda i: (i, 0)),
        out_shape=jax.ShapeDtypeStruct((N, 32), jnp.bfloat16),
    )(U, R)


def _decode_body(a_ref, b_ref, o_ref):
    z = lax.dot_general(a_ref[...], b_ref[...], (((1,), (1,)), ((), ())),
                        preferred_element_type=jnp.float32)
    o_ref[...] = 0.5 * jnp.tanh(0.5 * z) + 0.5


def _decode(Hf):
    bm, bn = 512, 2048
    return pl.pallas_call(
        _decode_body,
        grid=(pl.cdiv(N, bm), pl.cdiv(N, bn)),
        in_specs=[
            pl.BlockSpec((bm, 32), lambda i, j: (i, 0)),
            pl.BlockSpec((bn, 32), lambda i, j: (j, 0)),
        ],
        out_specs=pl.BlockSpec((bm, bn), lambda i, j: (i, j)),
        out_shape=jax.ShapeDtypeStruct((N, N), jnp.float32),
        compiler_params=pltpu.CompilerParams(
            dimension_semantics=("parallel", "parallel")),
    )(Hf, Hf)


def kernel(X, edge_index, A_values, W0, W1, v0_0, v1_0, v0_1, v1_1):
    H1, F1 = _encode1(X, W0, v0_0, v1_0)
    U1, RS1 = _edge_kernel_64(edge_index, A_values, H1, F1)
    H2, F2 = _combine_encode2(U1, RS1[:, 0, :], W1, v0_1, v1_1)
    U2, RS2 = _edge_kernel_32(edge_index, A_values, H2, F2)
    Hf = _combine(U2, RS2[:, 0, :])
    return _decode(Hf)


# speedup vs baseline: 38.6803x; 1.0772x over previous
import functools

import jax
import jax.numpy as jnp
from jax import lax
from jax.experimental import pallas as pl
from jax.experimental.pallas import tpu as pltpu
from jax.experimental.pallas import tpu_sc as plsc

N = 10000
E = 320000
NC = 2
NS = 16
NW = NC * NS
LANE = 16
CHUNK = 128
NCHUNKS = E // CHUNK
NPAD = 10240
ROWS_PER_TILE = NPAD // NS
ZROWS = 128

_SC_MESH = plsc.VectorSubcoreMesh(core_axis_name="c", subcore_axis_name="s")
_SC_PARAMS = pltpu.CompilerParams(needs_layout_passes=False,
                                  use_tc_tiling_on_sc=False)


def _make_edge_kernel(D):
    @functools.partial(
        pl.kernel,
        out_type=[
            jax.ShapeDtypeStruct((NC, NPAD, D), jnp.float32),
            jax.ShapeDtypeStruct((NC, 1, NPAD), jnp.float32),
        ],
        mesh=_SC_MESH,
        compiler_params=_SC_PARAMS,
        scratch_types=[
            pltpu.VMEM((2, N), jnp.float32),
            pltpu.VMEM((2, 2, CHUNK), jnp.int32),
            pltpu.VMEM((2, 1, CHUNK), jnp.int32),
            pltpu.VMEM((2, CHUNK), jnp.float32),
            pltpu.VMEM((2, CHUNK), jnp.float32),
            pltpu.VMEM((2, CHUNK, D), jnp.float32),
            pltpu.VMEM((2, CHUNK, D), jnp.float32),
            pltpu.VMEM((ZROWS, D), jnp.float32),
            pltpu.VMEM((ROWS_PER_TILE,), jnp.float32),
            pltpu.VMEM_SHARED((NPAD, D), jnp.float32),
            pltpu.VMEM_SHARED((NPAD,), jnp.float32),
            pltpu.SemaphoreType.DMA((2,)),
            pltpu.SemaphoreType.DMA((2,)),
            pltpu.SemaphoreType.DMA((2,)),
        ],
    )
    def edge_kernel(ei_hbm, a_hbm, h_hbm, f_hbm,
                    u_out, rs_out,
                    f_v, idx_v, sidx_v, a_v, w_v, rows_v, srows_v,
                    zrow_v, zrs_v, u_sh, rs_sh, sem_i, sem_g, sem_s):
        c = lax.axis_index("c")
        s = lax.axis_index("s")
        wid = c * NS + s
        nct = jnp.where(wid < NCHUNKS % NW, NCHUNKS // NW + 1, NCHUNKS // NW)

        zero16 = jnp.zeros((LANE,), jnp.float32)
        zrow16 = jnp.zeros((LANE,), jnp.int32)
        onerow16 = jnp.full((LANE,), 1, jnp.int32)

        def issue_idx(k, b):
            e0 = (wid + NW * k) * CHUNK
            pltpu.async_copy(ei_hbm.at[:, pl.ds(e0, CHUNK)], idx_v.at[b],
                             sem_i.at[b])
            pltpu.async_copy(a_hbm.at[pl.ds(e0, CHUNK)], a_v.at[b],
                             sem_i.at[b])

        def wait_idx(b):
            pltpu.make_async_copy(ei_hbm.at[:, pl.ds(0, CHUNK)], idx_v.at[b],
                                  sem_i.at[b]).wait()
            pltpu.make_async_copy(a_hbm.at[pl.ds(0, CHUNK)], a_v.at[b],
                                  sem_i.at[b]).wait()

        def issue_gather(b):
            pltpu.async_copy(h_hbm.at[idx_v.at[b, 1]], rows_v.at[b],
                             sem_g.at[b])

        def wait_gather(b):
            pltpu.make_async_copy(h_hbm.at[idx_v.at[b, 1]], rows_v.at[b],
                                  sem_g.at[b]).wait()

        def issue_scatter_rows(b):
            pltpu.async_copy(srows_v.at[b], u_sh.at[sidx_v.at[b, 0]],
                             sem_s.at[b], add=True)

        def wait_scatter_rows(b):
            pltpu.make_async_copy(srows_v.at[b], u_sh.at[sidx_v.at[b, 0]],
                                  sem_s.at[b]).wait()

        issue_idx(0, 0)
        issue_idx(1, 1)

        @pl.loop(0, ZROWS)
        def _(i):
            @pl.loop(0, D, step=LANE)
            def _(q):
                zrow_v[i, pl.ds(q, LANE)] = zero16

        @pl.loop(0, ROWS_PER_TILE, step=LANE)
        def _(q):
            zrs_v[pl.ds(q, LANE)] = zero16

        @pl.loop(0, ROWS_PER_TILE // ZROWS)
        def _(k):
            pltpu.sync_copy(zrow_v, u_sh.at[pl.ds(s * ROWS_PER_TILE + k * ZROWS, ZROWS)])

        pltpu.sync_copy(zrs_v, rs_sh.at[pl.ds(s * ROWS_PER_TILE, ROWS_PER_TILE)])

        pltpu.sync_copy(f_hbm, f_v)

        wait_idx(0)
        issue_gather(0)

        plsc.subcore_barrier()

        @pl.loop(0, NCHUNKS // NW + 1)
        def _(k):
            @pl.when(k < nct)
            def _():
                b = k & 1

                @pl.when(k + 1 < nct)
                def _():
                    wait_idx(1 - b)

                    @pl.when(k >= 1)
                    def _():
                        wait_scatter_rows(1 - b)

                    issue_gather(1 - b)

                for g in range(CHUNK // LANE):
                    s16 = idx_v[b, 0, pl.ds(g * LANE, LANE)]
                    d16 = idx_v[b, 1, pl.ds(g * LANE, LANE)]
                    a16 = a_v[b, pl.ds(g * LANE, LANE)]
                    sidx_v[b, 0, pl.ds(g * LANE, LANE)] = s16
                    f1g = plsc.load_gather(f_v, [zrow16, s16])
                    f2g = plsc.load_gather(f_v, [onerow16, d16])
                    x = a16 * (f1g + f2g)
                    att = 1.0 / (1.0 + jnp.exp(-x))
                    w16 = jnp.exp(att)
                    w_v[b, pl.ds(g * LANE, LANE)] = w16

                pltpu.sync_copy(w_v.at[b], rs_sh.at[sidx_v.at[b, 0]], add=True)

                wait_gather(b)

                for g in range(CHUNK // LANE):
                    w16 = w_v[b, pl.ds(g * LANE, LANE)]
                    for j in range(LANE):
                        wj = jnp.take(w16, jnp.full((LANE,), j, jnp.int32))
                        r = g * LANE + j
                        for q in range(D // LANE):
                            srows_v[b, r, pl.ds(q * LANE, LANE)] = (
                                rows_v[b, r, pl.ds(q * LANE, LANE)] * wj)

                issue_scatter_rows(b)

                @pl.when(k + 2 < nct)
                def _():
                    issue_idx(k + 2, b)

        wait_scatter_rows((nct - 1) & 1)

        @pl.when(nct >= 2)
        def _():
            wait_scatter_rows(nct & 1)

        plsc.subcore_barrier()

        @pl.loop(0, ROWS_PER_TILE // ZROWS)
        def _(k):
            r0 = s * ROWS_PER_TILE + k * ZROWS
            pltpu.sync_copy(u_sh.at[pl.ds(r0, ZROWS)], u_out.at[c, pl.ds(r0, ZROWS)])

        pltpu.sync_copy(rs_sh.at[pl.ds(s * ROWS_PER_TILE, ROWS_PER_TILE)],
                        rs_out.at[c, 0, pl.ds(s * ROWS_PER_TILE, ROWS_PER_TILE)])

    return edge_kernel


_edge_kernel_64 = _make_edge_kernel(64)
_edge_kernel_32 = _make_edge_kernel(32)


def _dot(a, b):
    return lax.dot_general(a, b, (((1,), (0,)), ((), ())),
                           preferred_element_type=jnp.float32)


def _frow(h, v):
    return lax.dot_general(v, h, (((0,), (1,)), ((), ())),
                           preferred_element_type=jnp.float32)


def _encode1_body(x_ref, w_ref, v0_ref, v1_ref, h_ref, f_ref):
    h = _dot(x_ref[...], w_ref[...])
    h_ref[...] = h
    f_ref[...] = jnp.concatenate(
        [_frow(h, v0_ref[...]), _frow(h, v1_ref[...])], axis=0)


def _encode1(X, W0, v0, v1):
    bm = 1024
    return pl.pallas_call(
        _encode1_body,
        grid=(pl.cdiv(N, bm),),
        in_specs=[
            pl.BlockSpec((bm, 128), lambda i: (i, 0)),
            pl.BlockSpec((128, 64), lambda i: (0, 0)),
            pl.BlockSpec((64, 1), lambda i: (0, 0)),
            pl.BlockSpec((64, 1), lambda i: (0, 0)),
        ],
        out_specs=[
            pl.BlockSpec((bm, 64), lambda i: (i, 0)),
            pl.BlockSpec((2, bm), lambda i: (0, i)),
        ],
        out_shape=[
            jax.ShapeDtypeStruct((N, 64), jnp.float32),
            jax.ShapeDtypeStruct((2, N), jnp.float32),
        ],
    )(X, W0, v0, v1)


def _combine_encode2_body(u_ref, r_ref, w_ref, v0_ref, v1_ref, h_ref, f_ref):
    rs = r_ref[0:1, :] + r_ref[1:2, :]
    rs = jnp.where(rs == 0.0, 1.0, rs)
    rs = jnp.transpose(rs, (1, 0))
    hin = (u_ref[0] + u_ref[1]) / rs
    h = _dot(hin, w_ref[...])
    h_ref[...] = h
    f_ref[...] = jnp.concatenate(
        [_frow(h, v0_ref[...]), _frow(h, v1_ref[...])], axis=0)


def _combine_encode2(U, R, W1, v0, v1):
    bm = 1024
    return pl.pallas_call(
        _combine_encode2_body,
        grid=(NPAD // bm,),
        in_specs=[
            pl.BlockSpec((2, bm, 64), lambda i: (0, i, 0)),
            pl.BlockSpec((2, bm), lambda i: (0, i)),
            pl.BlockSpec((64, 32), lambda i: (0, 0)),
            pl.BlockSpec((32, 1), lambda i: (0, 0)),
            pl.BlockSpec((32, 1), lambda i: (0, 0)),
        ],
        out_specs=[
            pl.BlockSpec((bm, 32), lambda i: (i, 0)),
            pl.BlockSpec((2, bm), lambda i: (0, i)),
        ],
        out_shape=[
            jax.ShapeDtypeStruct((N, 32), jnp.float32),
            jax.ShapeDtypeStruct((2, N), jnp.float32),
        ],
    )(U, R, W1, v0, v1)


def _combine_body(u_ref, r_ref, h_ref):
    rs = r_ref[0:1, :] + r_ref[1:2, :]
    rs = jnp.where(rs == 0.0, 1.0, rs)
    rs = jnp.transpose(rs, (1, 0))
    h_ref[...] = ((u_ref[0] + u_ref[1]) / rs).astype(jnp.bfloat16)


def _combine(U, R):
    bm = 1024
    return pl.pallas_call(
        _combine_body,
        grid=(NPAD // bm,),
        in_specs=[
            pl.BlockSpec((2, bm, 32), lambda i: (0, i, 0)),
            pl.BlockSpec((2, bm), lambda i: (0, i)),
        ],
        out_specs=pl.BlockSpec((bm, 32), lambda i: (i, 0)),
        out_shape=jax.ShapeDtypeStruct((N, 32), jnp.bfloat16),
    )(U, R)


def _decode_body(a_ref, b_ref, o_ref):
    z = lax.dot_general(a_ref[...], b_ref[...], (((1,), (1,)), ((), ())),
                        preferred_element_type=jnp.float32)
    o_ref[...] = 0.5 * jnp.tanh(0.5 * z) + 0.5


def _decode(Hf):
    bm, bn = 1024, 2048
    return pl.pallas_call(
        _decode_body,
        grid=(pl.cdiv(N, bm), pl.cdiv(N, bn)),
        in_specs=[
            pl.BlockSpec((bm, 32), lambda i, j: (i, 0)),
            pl.BlockSpec((bn, 32), lambda i, j: (j, 0)),
        ],
        out_specs=pl.BlockSpec((bm, bn), lambda i, j: (i, j)),
        out_shape=jax.ShapeDtypeStruct((N, N), jnp.float32),
        compiler_params=pltpu.CompilerParams(
            dimension_semantics=("parallel", "parallel")),
    )(Hf, Hf)


def kernel(X, edge_index, A_values, W0, W1, v0_0, v1_0, v0_1, v1_1):
    H1, F1 = _encode1(X, W0, v0_0, v1_0)
    U1, RS1 = _edge_kernel_64(edge_index, A_values, H1, F1)
    H2, F2 = _combine_encode2(U1, RS1[:, 0, :], W1, v0_1, v1_1)
    U2, RS2 = _edge_kernel_32(edge_index, A_values, H2, F2)
    Hf = _combine(U2, RS2[:, 0, :])
    return _decode(Hf)
